# baseline (speedup 1.0000x reference)
import functools

import jax
import jax.numpy as jnp
from jax import lax
from jax.experimental import pallas as pl
from jax.experimental.pallas import tpu as pltpu
from jax.experimental.pallas import tpu_sc as plsc

_D = 128
_B = 4096
_F = 26
_N = _B * _F
_V = 1000
_NC, _NS = 2, 16
_NW = _NC * _NS
_BPW = _N // _NW
_C = 256
_NCHUNK = _BPW // _C
_NB = 3

_mesh = plsc.VectorSubcoreMesh(core_axis_name="c", subcore_axis_name="s")


@functools.partial(
    pl.kernel,
    out_type=jax.ShapeDtypeStruct((_N, _D), jnp.float32),
    mesh=_mesh,
    compiler_params=pltpu.CompilerParams(use_tc_tiling_on_sc=True),
    scratch_types=[
        pltpu.VMEM((_BPW,), jnp.int32),
        pltpu.VMEM((_NB, _C, _D), jnp.float32),
        pltpu.VMEM_SHARED((_V, _D), jnp.float32),
        pltpu.SemaphoreType.DMA,
        pltpu.SemaphoreType.DMA,
        pltpu.SemaphoreType.DMA,
    ],
)
def _gather_kernel(idx_hbm, table_hbm, out_hbm, idx_v, rows_v, table_sh,
                   sem_g, sem_w, sem_r):
    sid = lax.axis_index("s")
    wid = sid * _NC + lax.axis_index("c")
    base = wid * _BPW

    def gather(j, src, sem):
        return pltpu.async_copy(
            src.at[idx_v.at[pl.ds(j * _C, _C)]], rows_v.at[j % _NB], sem)

    def writeback(j):
        return pltpu.async_copy(
            rows_v.at[j % _NB], out_hbm.at[pl.ds(base + j * _C, _C)], sem_w)

    pltpu.sync_copy(idx_hbm.at[pl.ds(base, _C)], idx_v.at[pl.ds(0, _C)])
    g = [None] * _NCHUNK
    w = [None] * _NCHUNK
    g[0] = gather(0, table_hbm, sem_r)
    pltpu.sync_copy(idx_hbm.at[pl.ds(base + _C, _BPW - _C)],
                    idx_v.at[pl.ds(_C, _BPW - _C)])
    g[1] = gather(1, table_hbm, sem_r)

    @pl.when(sid == 0)
    def _():
        pltpu.sync_copy(table_hbm, table_sh)

    plsc.subcore_barrier()

    for j in range(_NCHUNK):
        if j + _NB - 1 < _NCHUNK:
            if j >= 1:
                w[j - 1].wait()
            g[j + _NB - 1] = gather(j + _NB - 1, table_sh, sem_g)
        g[j].wait()
        w[j] = writeback(j)
    for j in range(_NCHUNK - _NB, _NCHUNK):
        w[j].wait()


def kernel(x, embed_matrix):
    idx = x.astype(jnp.int32).T.reshape(-1)
    out = _gather_kernel(idx, embed_matrix)
    return out.reshape(_F, _B, _D).transpose(1, 0, 2)


# speedup vs baseline: 1.1572x; 1.1572x over previous
import functools

import jax
import jax.numpy as jnp
from jax import lax
from jax.experimental import pallas as pl
from jax.experimental.pallas import tpu as pltpu
from jax.experimental.pallas import tpu_sc as plsc

_D = 128
_B = 4096
_F = 26
_N = _B * _F
_V = 1000
_NC, _NS = 2, 16
_NW = _NC * _NS
_BPW = _N // _NW
_C = 256
_NCHUNK = _BPW // _C
_NB = 3

_mesh = plsc.VectorSubcoreMesh(core_axis_name="c", subcore_axis_name="s")


@functools.partial(
    pl.kernel,
    out_type=jax.ShapeDtypeStruct((_N, _D), jnp.float32),
    mesh=_mesh,
    compiler_params=pltpu.CompilerParams(use_tc_tiling_on_sc=True),
    scratch_types=[
        pltpu.VMEM((_BPW,), jnp.int32),
        pltpu.VMEM((_NB, _C, _D), jnp.float32),
        pltpu.VMEM_SHARED((_V, _D), jnp.float32),
        pltpu.SemaphoreType.DMA,
        pltpu.SemaphoreType.DMA,
        pltpu.SemaphoreType.DMA,
    ],
)
def _gather_kernel(idx_hbm, table_hbm, out_hbm, idx_v, rows_v, table_sh,
                   sem_g, sem_w, sem_r):
    sid = lax.axis_index("s")
    wid = sid * _NC + lax.axis_index("c")
    base = wid * _BPW

    def gather(j, src, sem):
        return pltpu.async_copy(
            src.at[idx_v.at[pl.ds(j * _C, _C)]], rows_v.at[j % _NB], sem)

    def writeback(j):
        return pltpu.async_copy(
            rows_v.at[j % _NB], out_hbm.at[pl.ds(base + j * _C, _C)], sem_w)

    @pl.when(sid == 0)
    def _():
        pltpu.sync_copy(table_hbm, table_sh)

    pltpu.sync_copy(idx_hbm.at[pl.ds(base, _BPW)], idx_v)
    plsc.subcore_barrier()

    g = [None] * _NCHUNK
    w = [None] * _NCHUNK
    for j in range(_NB - 1):
        g[j] = gather(j, table_sh, sem_g)
    for j in range(_NCHUNK):
        if j + _NB - 1 < _NCHUNK:
            if j >= 1:
                w[j - 1].wait()
            g[j + _NB - 1] = gather(j + _NB - 1, table_sh, sem_g)
        g[j].wait()
        w[j] = writeback(j)
    for j in range(_NCHUNK - _NB, _NCHUNK):
        w[j].wait()


def kernel(x, embed_matrix):
    idx = x.astype(jnp.int32).T.reshape(-1)
    out = _gather_kernel(idx, embed_matrix)
    return out.reshape(_F, _B, _D).transpose(1, 0, 2)


# speedup vs baseline: 1.1718x; 1.0126x over previous
import functools

import jax
import jax.numpy as jnp
from jax import lax
from jax.experimental import pallas as pl
from jax.experimental.pallas import tpu as pltpu
from jax.experimental.pallas import tpu_sc as plsc

_D = 128
_B = 4096
_F = 26
_N = _B * _F
_V = 1000
_NC, _NS = 2, 16
_NW = _NC * _NS
_BPW = _N // _NW
_C = 256
_NCHUNK = _BPW // _C
_NB = 3

_mesh = plsc.VectorSubcoreMesh(core_axis_name="c", subcore_axis_name="s")


@functools.partial(
    pl.kernel,
    out_type=jax.ShapeDtypeStruct((_N, _D), jnp.float32),
    mesh=_mesh,
    compiler_params=pltpu.CompilerParams(use_tc_tiling_on_sc=True),
    scratch_types=[
        pltpu.VMEM((_BPW,), jnp.int32),
        pltpu.VMEM((_NB, _C, _D), jnp.float32),
        pltpu.VMEM_SHARED((_V, _D), jnp.float32),
        pltpu.SemaphoreType.DMA,
        pltpu.SemaphoreType.DMA,
        pltpu.SemaphoreType.DMA,
    ],
)
def _gather_kernel(idx_hbm, table_hbm, out_hbm, idx_v, rows_v, table_sh,
                   sem_g, sem_w, sem_r):
    sid = lax.axis_index("s")
    wid = sid * _NC + lax.axis_index("c")
    base = wid * _BPW

    def gather(j, src, sem):
        return pltpu.async_copy(
            src.at[idx_v.at[pl.ds(j * _C, _C)]], rows_v.at[j % _NB], sem)

    def writeback(j):
        return pltpu.async_copy(
            rows_v.at[j % _NB], out_hbm.at[pl.ds(base + j * _C, _C)], sem_w)

    @pl.when(sid == 0)
    def _():
        pltpu.async_copy(table_hbm, table_sh, sem_r)

    pltpu.sync_copy(idx_hbm.at[pl.ds(base, _BPW)], idx_v)

    @pl.when(sid == 0)
    def _():
        pltpu.make_async_copy(table_hbm, table_sh, sem_r).wait()

    plsc.subcore_barrier()

    g = [None] * _NCHUNK
    w = [None] * _NCHUNK
    for j in range(_NB - 1):
        g[j] = gather(j, table_sh, sem_g)
    for j in range(_NCHUNK):
        if j + _NB - 1 < _NCHUNK:
            if j >= 1:
                w[j - 1].wait()
            g[j + _NB - 1] = gather(j + _NB - 1, table_sh, sem_g)
        g[j].wait()
        w[j] = writeback(j)
    for j in range(_NCHUNK - _NB, _NCHUNK):
        w[j].wait()


def kernel(x, embed_matrix):
    idx = x.astype(jnp.int32).T.reshape(-1)
    out = _gather_kernel(idx, embed_matrix)
    return out.reshape(_F, _B, _D).transpose(1, 0, 2)
